# baseline (speedup 1.0000x reference)
import jax
import jax.numpy as jnp
from jax.experimental import pallas as pl
from jax.experimental.pallas import tpu as pltpu

N_EXPERTS = 64
TOP_K = 8
N_GROUP = 8
GROUP_SIZE = N_EXPERTS // N_GROUP
TOPK_GROUP = 4
SCALE = 2.5
NEG = -1e30


def _routing_kernel(x_ref, w_ref, b_ref, idx_ref, wgt_ref):
    x = x_ref[...]
    w = w_ref[...]
    b = b_ref[...]
    logits = jax.lax.dot_general(
        w, x, (((1,), (1,)), ((), ())), preferred_element_type=jnp.float32)
    scores = jax.nn.sigmoid(logits)
    sfc = scores + b
    B = sfc.shape[1]

    x3 = sfc.reshape(N_GROUP, GROUP_SIZE, B)
    i1 = jax.lax.broadcasted_iota(jnp.int32, (N_GROUP, GROUP_SIZE, B), 1)
    m1 = jnp.max(x3, axis=1)
    idx1 = jnp.min(jnp.where(x3 == m1[:, None, :], i1, GROUP_SIZE), axis=1)
    masked = jnp.where(i1 == idx1[:, None, :], NEG, x3)
    m2 = jnp.max(masked, axis=1)
    gs = m1 + m2

    giota = jax.lax.broadcasted_iota(jnp.int32, (N_GROUP, B), 0)
    gmask = jnp.zeros((N_GROUP, B), dtype=jnp.bool_)
    for _ in range(TOPK_GROUP):
        gm = jnp.max(gs, axis=0)
        gidx = jnp.min(jnp.where(gs == gm[None, :], giota, N_GROUP), axis=0)
        sel = giota == gidx[None, :]
        gmask = jnp.logical_or(gmask, sel)
        gs = jnp.where(sel, NEG, gs)

    emask = jnp.broadcast_to(
        gmask[:, None, :], (N_GROUP, GROUP_SIZE, B)).reshape(N_EXPERTS, B)
    tmp = jnp.where(emask, sfc, 0.0)

    eiota = jax.lax.broadcasted_iota(jnp.int32, (N_EXPERTS, B), 0)
    for k in range(TOP_K):
        m = jnp.max(tmp, axis=0)
        ei = jnp.min(jnp.where(tmp == m[None, :], eiota, N_EXPERTS), axis=0)
        sel = eiota == ei[None, :]
        wv = jnp.sum(jnp.where(sel, scores, 0.0), axis=0)
        idx_ref[pl.ds(k, 1), :] = ei[None, :].astype(jnp.int32)
        wgt_ref[pl.ds(k, 1), :] = wv[None, :]
        if k < TOP_K - 1:
            tmp = jnp.where(sel, -1.0, tmp)

    w8 = wgt_ref[...]
    denom = jnp.sum(w8, axis=0, keepdims=True) + 1e-20
    wgt_ref[...] = w8 * (SCALE / denom)


def kernel(hidden_states, weight, e_score_correction_bias):
    bsz, seq_len, h = hidden_states.shape
    n_tok = bsz * seq_len
    hs = hidden_states.reshape(n_tok, h)
    bias = e_score_correction_bias.reshape(N_EXPERTS, 1)
    BLK = 2048
    grid = (n_tok // BLK,)
    idx_t, wgt_t = pl.pallas_call(
        _routing_kernel,
        grid=grid,
        in_specs=[
            pl.BlockSpec((BLK, h), lambda i: (i, 0)),
            pl.BlockSpec((N_EXPERTS, h), lambda i: (0, 0)),
            pl.BlockSpec((N_EXPERTS, 1), lambda i: (0, 0)),
        ],
        out_specs=[
            pl.BlockSpec((TOP_K, BLK), lambda i: (0, i)),
            pl.BlockSpec((TOP_K, BLK), lambda i: (0, i)),
        ],
        out_shape=[
            jax.ShapeDtypeStruct((TOP_K, n_tok), jnp.int32),
            jax.ShapeDtypeStruct((TOP_K, n_tok), jnp.float32),
        ],
        compiler_params=pltpu.CompilerParams(
            dimension_semantics=("parallel",)),
    )(hs, weight, bias)
    return (idx_t.T, wgt_t.T)


# speedup vs baseline: 1.0843x; 1.0843x over previous
import jax
import jax.numpy as jnp
from jax.experimental import pallas as pl
from jax.experimental.pallas import tpu as pltpu

N_EXPERTS = 64
TOP_K = 8
N_GROUP = 8
GROUP_SIZE = N_EXPERTS // N_GROUP
TOPK_GROUP = 4
SCALE = 2.5
NEG = -1e30


def _routing_kernel(x_ref, w_ref, b_ref, idx_ref, wgt_ref):
    x = x_ref[...]
    w = w_ref[...]
    b = b_ref[...]
    logits = jax.lax.dot_general(
        w, x, (((1,), (1,)), ((), ())), preferred_element_type=jnp.float32)
    scores = jax.nn.sigmoid(logits)
    sfc = scores + b
    B = sfc.shape[1]

    x3 = sfc.reshape(N_GROUP, GROUP_SIZE, B)
    i1 = jax.lax.broadcasted_iota(jnp.int32, (N_GROUP, GROUP_SIZE, B), 1)
    m1 = jnp.max(x3, axis=1)
    idx1 = jnp.argmax(x3, axis=1)
    masked = jnp.where(i1 == idx1[:, None, :], NEG, x3)
    m2 = jnp.max(masked, axis=1)
    gs = m1 + m2

    giota = jax.lax.broadcasted_iota(jnp.int32, (N_GROUP, B), 0)
    gmask = jnp.zeros((N_GROUP, B), dtype=jnp.bool_)
    for _ in range(TOPK_GROUP):
        gidx = jnp.argmax(gs, axis=0)
        sel = giota == gidx[None, :]
        gmask = jnp.logical_or(gmask, sel)
        gs = jnp.where(sel, NEG, gs)

    emask = jnp.broadcast_to(
        gmask[:, None, :], (N_GROUP, GROUP_SIZE, B)).reshape(N_EXPERTS, B)
    tmp = jnp.where(emask, sfc, 0.0)

    eiota = jax.lax.broadcasted_iota(jnp.int32, (N_EXPERTS, B), 0)
    idxs, wgts = [], []
    for k in range(TOP_K):
        ei = jnp.argmax(tmp, axis=0)
        sel = eiota == ei[None, :]
        wv = jnp.sum(jnp.where(sel, scores, 0.0), axis=0)
        idxs.append(ei)
        wgts.append(wv)
        if k < TOP_K - 1:
            tmp = jnp.where(sel, -1.0, tmp)

    topk_i = jnp.stack(idxs, axis=0)
    topk_w = jnp.stack(wgts, axis=0)
    denom = jnp.sum(topk_w, axis=0, keepdims=True) + 1e-20
    idx_ref[...] = topk_i.astype(jnp.int32)
    wgt_ref[...] = topk_w * (SCALE / denom)


def kernel(hidden_states, weight, e_score_correction_bias):
    bsz, seq_len, h = hidden_states.shape
    n_tok = bsz * seq_len
    hs = hidden_states.reshape(n_tok, h)
    bias = e_score_correction_bias.reshape(N_EXPERTS, 1)
    BLK = 2048
    grid = (n_tok // BLK,)
    idx_t, wgt_t = pl.pallas_call(
        _routing_kernel,
        grid=grid,
        in_specs=[
            pl.BlockSpec((BLK, h), lambda i: (i, 0)),
            pl.BlockSpec((N_EXPERTS, h), lambda i: (0, 0)),
            pl.BlockSpec((N_EXPERTS, 1), lambda i: (0, 0)),
        ],
        out_specs=[
            pl.BlockSpec((TOP_K, BLK), lambda i: (0, i)),
            pl.BlockSpec((TOP_K, BLK), lambda i: (0, i)),
        ],
        out_shape=[
            jax.ShapeDtypeStruct((TOP_K, n_tok), jnp.int32),
            jax.ShapeDtypeStruct((TOP_K, n_tok), jnp.float32),
        ],
        compiler_params=pltpu.CompilerParams(
            dimension_semantics=("parallel",)),
    )(hs, weight, bias)
    return (idx_t.T, wgt_t.T)
